# Initial kernel scaffold; baseline (speedup 1.0000x reference)
import jax
import jax.numpy as jnp
from jax.experimental import pallas as pl


def kernel(input, scale):
    raise NotImplementedError("write your pallas kernel here")


# speedup vs baseline: 470.9925x; 470.9925x over previous
import jax
import jax.numpy as jnp
import numpy as np
from jax.experimental import pallas as pl
from jax.experimental.pallas import tpu as pltpu

K = 101
PAD = K // 2
RB = 56
SLAB = 160
ROWS = 216
LANES = 256


def _dilate_kernel(scale_ref, pt_ref, out_ref):
    r0 = pl.program_id(0) * RB
    inv4s = -0.25 / scale_ref[0, 0]

    slab = pt_ref[pl.ds(r0, SLAB), :]

    g2 = jnp.full((RB, LANES), -np.inf, jnp.float32)
    for j in range(K):
        w = float((j - PAD) ** 2) * inv4s
        g2 = jnp.maximum(g2, slab[j:j + RB, :] + w)

    acc = jnp.full((RB, 128), -np.inf, jnp.float32)
    for i in range(K):
        w = float((i - PAD) ** 2) * inv4s
        acc = jnp.maximum(acc, g2[:, i:i + 128] + w)
    out_ref[:, :] = acc[:, :K]


def kernel(input, scale):
    pt = jnp.full((ROWS, LANES), -np.inf, jnp.float32)
    pt = jax.lax.dynamic_update_slice(pt, input.T.astype(jnp.float32),
                                      (PAD, PAD))
    scale2 = jnp.reshape(scale, (1, 1)).astype(jnp.float32)
    return pl.pallas_call(
        _dilate_kernel,
        grid=(2,),
        in_specs=[
            pl.BlockSpec(memory_space=pltpu.SMEM),
            pl.BlockSpec((ROWS, LANES), lambda p: (0, 0)),
        ],
        out_specs=pl.BlockSpec((RB, K), lambda p: (p, 0)),
        out_shape=jax.ShapeDtypeStruct((K, K), jnp.float32),
        compiler_params=pltpu.CompilerParams(
            dimension_semantics=("parallel",)),
    )(scale2, pt)


# speedup vs baseline: 548.2962x; 1.1641x over previous
import jax
import jax.numpy as jnp
import numpy as np
from jax.experimental import pallas as pl
from jax.experimental.pallas import tpu as pltpu

K = 101
PAD = K // 2
RB = 56
IMG_R0 = 56
SLAB = 168
CROWS = 224
LANES = 256
NEG = float(-np.inf)


def _dilate_kernel(scale_ref, x_ref, out_ref, canvas_ref):
    r0 = pl.program_id(0) * RB
    inv4s = -0.25 / scale_ref[0, 0]

    xt = x_ref[:, :].T
    xt = jnp.concatenate([xt, jnp.full((3, K), NEG)], axis=0)
    blk = jnp.concatenate([jnp.full((104, PAD), NEG), xt,
                           jnp.full((104, LANES - PAD - K), NEG)], axis=1)
    canvas_ref[0:IMG_R0, :] = jnp.full((IMG_R0, LANES), NEG)
    canvas_ref[IMG_R0:IMG_R0 + 104, :] = blk
    canvas_ref[IMG_R0 + 104:CROWS, :] = jnp.full(
        (CROWS - IMG_R0 - 104, LANES), NEG)

    slab = canvas_ref[pl.ds(r0, SLAB), :]

    g2 = jnp.full((RB, LANES), NEG, jnp.float32)
    for j in range(K):
        w = float((j - PAD) ** 2) * inv4s
        g2 = jnp.maximum(g2, slab[j + 6:j + 6 + RB, :] + w)

    acc = jnp.full((RB, 128), NEG, jnp.float32)
    for i in range(K):
        w = float((i - PAD) ** 2) * inv4s
        acc = jnp.maximum(acc, g2[:, i:i + 128] + w)
    out_ref[:, :] = acc[:, :K]


def kernel(input, scale):
    scale2 = jnp.reshape(scale, (1, 1)).astype(jnp.float32)
    return pl.pallas_call(
        _dilate_kernel,
        grid=(2,),
        in_specs=[
            pl.BlockSpec(memory_space=pltpu.SMEM),
            pl.BlockSpec((K, K), lambda p: (0, 0)),
        ],
        out_specs=pl.BlockSpec((RB, K), lambda p: (p, 0)),
        out_shape=jax.ShapeDtypeStruct((K, K), jnp.float32),
        scratch_shapes=[pltpu.VMEM((CROWS, LANES), jnp.float32)],
        compiler_params=pltpu.CompilerParams(
            dimension_semantics=("parallel",)),
    )(scale2, input)


# speedup vs baseline: 633.1407x; 1.1547x over previous
import jax
import jax.numpy as jnp
import numpy as np
from jax.experimental import pallas as pl
from jax.experimental.pallas import tpu as pltpu

K = 101
PAD = K // 2
RB = 56
IMG_R0 = 56
SLAB = 168
CROWS = 224
LANES = 256
NEG = float(-np.inf)


def _dilate_kernel(scale_ref, x_ref, out_ref, canvas_ref):
    r0 = pl.program_id(0) * RB
    inv4s = -0.25 / scale_ref[0, 0]

    xt = x_ref[:, :].T
    xt = jnp.concatenate([xt, jnp.full((3, K), NEG)], axis=0)
    blk = jnp.concatenate([jnp.full((104, PAD), NEG), xt,
                           jnp.full((104, LANES - PAD - K), NEG)], axis=1)
    canvas_ref[0:IMG_R0, :] = jnp.full((IMG_R0, LANES), NEG)
    canvas_ref[IMG_R0:IMG_R0 + 104, :] = blk
    canvas_ref[IMG_R0 + 104:CROWS, :] = jnp.full(
        (CROWS - IMG_R0 - 104, LANES), NEG)

    slab = canvas_ref[pl.ds(r0, SLAB), :]

    g2 = slab[PAD + 6:PAD + 6 + RB, :]
    for j in range(PAD):
        w = float((j - PAD) ** 2) * inv4s
        m = jnp.maximum(slab[j + 6:j + 6 + RB, :],
                        slab[K + 5 - j:K + 5 - j + RB, :])
        g2 = jnp.maximum(g2, m + w)

    acc = pltpu.roll(g2, LANES - PAD, axis=1)[:, :128]
    for i in range(PAD):
        w = float((i - PAD) ** 2) * inv4s
        m = jnp.maximum(pltpu.roll(g2, LANES - i, axis=1),
                        pltpu.roll(g2, LANES - (K - 1) + i, axis=1))[:, :128]
        acc = jnp.maximum(acc, m + w)
    out_ref[:, :] = acc[:, :K]


def kernel(input, scale):
    scale2 = jnp.reshape(scale, (1, 1)).astype(jnp.float32)
    return pl.pallas_call(
        _dilate_kernel,
        grid=(2,),
        in_specs=[
            pl.BlockSpec(memory_space=pltpu.SMEM),
            pl.BlockSpec((K, K), lambda p: (0, 0)),
        ],
        out_specs=pl.BlockSpec((RB, K), lambda p: (p, 0)),
        out_shape=jax.ShapeDtypeStruct((K, K), jnp.float32),
        scratch_shapes=[pltpu.VMEM((CROWS, LANES), jnp.float32)],
        compiler_params=pltpu.CompilerParams(
            dimension_semantics=("parallel",)),
    )(scale2, input)


# speedup vs baseline: 1078.9514x; 1.7041x over previous
import jax
import jax.numpy as jnp
import numpy as np
from jax.experimental import pallas as pl
from jax.experimental.pallas import tpu as pltpu

K = 101
PAD = K // 2
RB = 112
IMG_R0 = 56
CROWS = 224
LANES = 256
CB = 104
NEG = float(-np.inf)


def _dilate_kernel(scale_ref, x_ref, out_ref, canvas_ref):
    inv4s = -0.25 / scale_ref[0, 0]

    xt = x_ref[:, :].T
    xt = jnp.concatenate([xt, jnp.full((3, K), NEG)], axis=0)
    blk = jnp.concatenate([jnp.full((104, PAD), NEG), xt,
                           jnp.full((104, LANES - PAD - K), NEG)], axis=1)
    canvas_ref[0:IMG_R0, :] = jnp.full((IMG_R0, LANES), NEG)
    canvas_ref[IMG_R0:IMG_R0 + 104, :] = blk
    canvas_ref[IMG_R0 + 104:CROWS, :] = jnp.full(
        (CROWS - IMG_R0 - 104, LANES), NEG)

    cv = canvas_ref[:, :]

    g2 = cv[PAD + 6:PAD + 6 + RB, :]
    for j in range(PAD):
        w = float((j - PAD) ** 2) * inv4s
        m = jnp.maximum(cv[j + 6:j + 6 + RB, :],
                        cv[K + 5 - j:K + 5 - j + RB, :])
        g2 = jnp.maximum(g2, m + w)

    g2t = g2.T

    acc = g2t[PAD:PAD + CB, :]
    for i in range(PAD):
        w = float((i - PAD) ** 2) * inv4s
        m = jnp.maximum(g2t[i:i + CB, :],
                        g2t[K - 1 - i:K - 1 - i + CB, :])
        acc = jnp.maximum(acc, m + w)

    out_ref[:, :] = acc.T[:K, :K]


def kernel(input, scale):
    scale2 = jnp.reshape(scale, (1, 1)).astype(jnp.float32)
    return pl.pallas_call(
        _dilate_kernel,
        in_specs=[
            pl.BlockSpec(memory_space=pltpu.SMEM),
            pl.BlockSpec((K, K), lambda p: (0, 0)),
        ],
        out_specs=pl.BlockSpec((K, K), lambda p: (0, 0)),
        out_shape=jax.ShapeDtypeStruct((K, K), jnp.float32),
        scratch_shapes=[pltpu.VMEM((CROWS, LANES), jnp.float32)],
        grid=(1,),
        compiler_params=pltpu.CompilerParams(
            dimension_semantics=("arbitrary",)),
    )(scale2, input)


# speedup vs baseline: 1286.2082x; 1.1921x over previous
import jax
import jax.numpy as jnp
import numpy as np
from jax.experimental import pallas as pl
from jax.experimental.pallas import tpu as pltpu

K = 101
PAD = K // 2
RB = 112
IMG_R0 = 56
CROWS = 224
LANES = 256
CB = 104
SH1R = 216
SH2R = 208
NEG = float(-np.inf)


def _dilate_kernel(scale_ref, x_ref, out_ref, canvas_ref, sh1_ref, sh2_ref):
    inv4s = -0.25 / scale_ref[0, 0]

    xt = x_ref[:, :].T
    xt = jnp.concatenate([xt, jnp.full((3, K), NEG)], axis=0)
    blk = jnp.concatenate([jnp.full((104, PAD), NEG), xt,
                           jnp.full((104, LANES - PAD - K), NEG)], axis=1)
    canvas_ref[0:IMG_R0, :] = jnp.full((IMG_R0, LANES), NEG)
    canvas_ref[IMG_R0:IMG_R0 + 104, :] = blk
    canvas_ref[IMG_R0 + 104:CROWS, :] = jnp.full(
        (CROWS - IMG_R0 - 104, LANES), NEG)

    cv = canvas_ref[:, :]

    for b in range(8):
        sh1_ref[b, :, :] = cv[b:b + SH1R, :]

    def tap1(j):
        t0 = j + IMG_R0 - PAD
        return sh1_ref[t0 % 8, t0 - t0 % 8:t0 - t0 % 8 + RB, :]

    g2 = tap1(PAD)
    for j in range(PAD):
        w = float((j - PAD) ** 2) * inv4s
        m = jnp.maximum(tap1(j), tap1(K - 1 - j))
        g2 = jnp.maximum(g2, m + w)

    g2t = g2.T
    for b in range(8):
        sh2_ref[b, :, :] = g2t[b:b + SH2R, :]

    def tap2(i):
        return sh2_ref[i % 8, i - i % 8:i - i % 8 + CB, :]

    acc = tap2(PAD)
    for i in range(PAD):
        w = float((i - PAD) ** 2) * inv4s
        m = jnp.maximum(tap2(i), tap2(K - 1 - i))
        acc = jnp.maximum(acc, m + w)

    out_ref[:, :] = acc.T[:K, :K]


def kernel(input, scale):
    scale2 = jnp.reshape(scale, (1, 1)).astype(jnp.float32)
    return pl.pallas_call(
        _dilate_kernel,
        in_specs=[
            pl.BlockSpec(memory_space=pltpu.SMEM),
            pl.BlockSpec((K, K), lambda p: (0, 0)),
        ],
        out_specs=pl.BlockSpec((K, K), lambda p: (0, 0)),
        out_shape=jax.ShapeDtypeStruct((K, K), jnp.float32),
        scratch_shapes=[
            pltpu.VMEM((CROWS, LANES), jnp.float32),
            pltpu.VMEM((8, SH1R, LANES), jnp.float32),
            pltpu.VMEM((8, SH2R, RB), jnp.float32),
        ],
        grid=(1,),
        compiler_params=pltpu.CompilerParams(
            dimension_semantics=("arbitrary",)),
    )(scale2, input)


# speedup vs baseline: 1313.4712x; 1.0212x over previous
import jax
import jax.numpy as jnp
import numpy as np
from jax.experimental import pallas as pl
from jax.experimental.pallas import tpu as pltpu

K = 101
PAD = K // 2
RB = 104
IMG_R0 = 56
CROWS = 224
LANES = 256
CB = 104
SH1R = 216
SH2R = 200
NEG = float(-np.inf)


def _dilate_kernel(scale_ref, x_ref, out_ref, sh1_ref, sh2_ref):
    inv4s = -0.25 / scale_ref[0, 0]

    xt = x_ref[:, :].T
    xt = jnp.concatenate([xt, jnp.full((3, K), NEG)], axis=0)
    blk = jnp.concatenate([jnp.full((104, PAD), NEG), xt,
                           jnp.full((104, LANES - PAD - K), NEG)], axis=1)
    cv = jnp.concatenate([jnp.full((IMG_R0, LANES), NEG), blk,
                          jnp.full((CROWS - IMG_R0 - 104, LANES), NEG)],
                         axis=0)

    for b in range(8):
        sh1_ref[b, :, :] = cv[b:b + SH1R, :]

    def tap1(j):
        t0 = j + IMG_R0 - PAD
        return sh1_ref[t0 % 8, t0 - t0 % 8:t0 - t0 % 8 + RB, :]

    g2 = tap1(PAD)
    for j in range(PAD):
        w = float((j - PAD) ** 2) * inv4s
        m = jnp.maximum(tap1(j), tap1(K - 1 - j))
        g2 = jnp.maximum(g2, m + w)

    g2t = g2.T
    for b in range(8):
        sh2_ref[b, :, :] = g2t[b:b + SH2R, :]

    def tap2(i):
        return sh2_ref[i % 8, i - i % 8:i - i % 8 + CB, :]

    acc = tap2(PAD)
    for i in range(PAD):
        w = float((i - PAD) ** 2) * inv4s
        m = jnp.maximum(tap2(i), tap2(K - 1 - i))
        acc = jnp.maximum(acc, m + w)

    out_ref[:, :] = acc.T[:K, :K]


def kernel(input, scale):
    scale2 = jnp.reshape(scale, (1, 1)).astype(jnp.float32)
    return pl.pallas_call(
        _dilate_kernel,
        in_specs=[
            pl.BlockSpec(memory_space=pltpu.SMEM),
            pl.BlockSpec((K, K), lambda p: (0, 0)),
        ],
        out_specs=pl.BlockSpec((K, K), lambda p: (0, 0)),
        out_shape=jax.ShapeDtypeStruct((K, K), jnp.float32),
        scratch_shapes=[
            pltpu.VMEM((8, SH1R, LANES), jnp.float32),
            pltpu.VMEM((8, SH2R, RB), jnp.float32),
        ],
        grid=(1,),
        compiler_params=pltpu.CompilerParams(
            dimension_semantics=("arbitrary",)),
    )(scale2, input)
